# baseline (speedup 1.0000x reference)
import functools

import jax
import jax.numpy as jnp
from jax import lax
from jax.experimental import pallas as pl
from jax.experimental.pallas import tpu as pltpu
from jax.experimental.pallas import tpu_sc as plsc

N_U = 5000
N_I = 5000
N = N_U + N_I
NPAD = 10240
E = 320000
D = 128
NC, NS = 2, 16
CH = 80
EPC = E // NC
EPTM = EPC // NS
NCH = EPTM // CH
EPT = E // NS
SEG = NPAD // NS
LANES = 16

_MESH = dict(core_axis_name="c", subcore_axis_name="s", num_cores=NC,
             num_subcores=NS)
_SC_PARAMS = pltpu.CompilerParams(needs_layout_passes=False)


def _rsqrt_newton(d):
    i = plsc.bitcast(d, jnp.int32)
    i = jnp.int32(0x5F3759DF) - lax.shift_right_logical(i, 1)
    y = plsc.bitcast(i, jnp.float32)
    for _ in range(3):
        y = y * (1.5 - 0.5 * d * y * y)
    return y


def _deg_body(src_hbm, dst_hbm, out_hbm, idx_v, hist_v, red_v, res_v,
              stage_sh, sem_p):
    cid = lax.axis_index("c")
    sid = lax.axis_index("s")
    zero16 = jnp.zeros((LANES,), jnp.float32)
    ones16 = jnp.ones((LANES,), jnp.float32)
    base = sid * EPT

    def _histogram(arr_hbm, h):
        pltpu.async_copy(arr_hbm.at[pl.ds(base, EPT)], idx_v, sem_p)

        def fill_zeros(i, _):
            hist_v[pl.ds(i * LANES, LANES)] = zero16
            return 0

        lax.fori_loop(0, NPAD // LANES, fill_zeros, 0)
        pltpu.make_async_copy(arr_hbm.at[pl.ds(base, EPT)], idx_v,
                              sem_p).wait()

        def hist(i, _):
            iv = idx_v[pl.ds(i * LANES, LANES)]
            plsc.addupdate_scatter(hist_v, [iv], ones16)
            return 0

        lax.fori_loop(0, EPT // LANES, hist, 0)

        pltpu.sync_copy(hist_v, stage_sh.at[sid])
        plsc.subcore_barrier()
        pltpu.sync_copy(stage_sh.at[:, pl.ds(sid * SEG, SEG)], red_v)

        def reduce(v, _):
            sl = pl.ds(v * LANES, LANES)
            acc = zero16
            for t in range(NS):
                acc = acc + red_v[t, sl]
            res_v[sl] = _rsqrt_newton(jnp.maximum(acc, 1.0))
            return 0

        lax.fori_loop(0, SEG // LANES, reduce, 0)
        pltpu.sync_copy(res_v, out_hbm.at[h, pl.ds(sid * SEG, SEG)])

    @pl.when(cid == 0)
    def _():
        _histogram(src_hbm, 0)

    @pl.when(cid == 1)
    def _():
        _histogram(dst_hbm, 1)


_deg_call = pl.kernel(
    _deg_body,
    out_type=jax.ShapeDtypeStruct((2, NPAD), jnp.float32),
    mesh=plsc.VectorSubcoreMesh(**_MESH),
    compiler_params=_SC_PARAMS,
    scratch_types=[
        pltpu.VMEM((EPT,), jnp.int32),
        pltpu.VMEM((NPAD,), jnp.float32),
        pltpu.VMEM((NS, SEG), jnp.float32),
        pltpu.VMEM((SEG,), jnp.float32),
        pltpu.VMEM_SHARED((NS, NPAD), jnp.float32),
        pltpu.SemaphoreType.DMA,
    ],
)


BR = 1000
NBU = N_U // BR


def _lin_body(u_ref, i_ref, wu_ref, wi_ref, bu_ref, bi_ref, r_ref, o_ref):
    j = pl.program_id(0)
    x = jnp.where(j < NBU, u_ref[...], i_ref[...])
    w = jnp.where(j < NBU, wu_ref[...], wi_ref[...])
    b = jnp.where(j < NBU, bu_ref[...], bi_ref[...])
    y = jnp.dot(x, w, preferred_element_type=jnp.float32) + b
    o_ref[...] = y * r_ref[...]


_lin_call = pl.pallas_call(
    _lin_body,
    grid=(N // BR,),
    in_specs=[
        pl.BlockSpec((BR, D), lambda j: (jnp.minimum(j, NBU - 1), 0)),
        pl.BlockSpec((BR, D), lambda j: (jnp.maximum(j - NBU, 0), 0)),
        pl.BlockSpec((D, D), lambda j: (0, 0)),
        pl.BlockSpec((D, D), lambda j: (0, 0)),
        pl.BlockSpec((1, D), lambda j: (0, 0)),
        pl.BlockSpec((1, D), lambda j: (0, 0)),
        pl.BlockSpec((BR, 1), lambda j: (j, 0)),
    ],
    out_specs=pl.BlockSpec((BR, D), lambda j: (j, 0)),
    out_shape=jax.ShapeDtypeStruct((N, D), jnp.float32),
)


NBUF = 2


def _msg_body(node_hbm, src_hbm, dst3_hbm, ef_hbm, out_hbm,
              sidx_v, didx3_v,
              ef0, ef1,
              rows0, rows1,
              acc_sh, sem_g, sem_e, sem_s, sem_p):
    efs = (ef0, ef1)
    rows = (rows0, rows1)
    cid = lax.axis_index("c")
    sid = lax.axis_index("s")
    base = cid * EPC + sid * EPTM
    chunk0 = base // CH

    pltpu.async_copy(src_hbm.at[pl.ds(base, EPTM)], sidx_v, sem_p)
    pltpu.async_copy(dst3_hbm.at[pl.ds(chunk0, NCH)], didx3_v,
                     sem_g.at[1])

    def issue(j, b):
        pltpu.async_copy(ef_hbm.at[pl.ds(base + j * CH, CH)],
                         efs[b], sem_e.at[b])
        pltpu.async_copy(node_hbm.at[sidx_v.at[pl.ds(j * CH, CH)]],
                         rows[b], sem_g.at[b])

    def consume(j, b, o, last):
        pltpu.make_async_copy(node_hbm.at[sidx_v.at[pl.ds(j * CH, CH)]],
                              rows[b], sem_g.at[b]).wait()
        pltpu.make_async_copy(ef_hbm.at[pl.ds(base + j * CH, CH)],
                              efs[b], sem_e.at[b]).wait()

        @plsc.parallel_loop(0, CH, 1, unroll=4)
        def scale(i):
            e = plsc.load_gather(efs[b],
                                 [jnp.full((LANES,), i, jnp.int32)])
            for k in range(D // LANES):
                sl = pl.ds(k * LANES, LANES)
                rows[b][i, sl] = rows[b][i, sl] * e

        @pl.when(j >= 1)
        def _():
            pltpu.make_async_copy(rows[o], acc_sh.at[didx3_v.at[j, 0]],
                                  sem_s.at[o]).wait()
        pltpu.async_copy(rows[b], acc_sh.at[didx3_v.at[j, 0]], sem_s.at[b],
                         add=True)
        if not last:
            issue(j + 1, o)

    def zero_rows(i, _):
        for k in range(D // LANES):
            rows1[i, pl.ds(k * LANES, LANES)] = jnp.zeros((LANES,),
                                                          jnp.float32)
        return 0

    lax.fori_loop(0, CH, zero_rows, 0)
    pltpu.make_async_copy(src_hbm.at[pl.ds(base, EPTM)], sidx_v,
                          sem_p).wait()
    issue(0, 0)
    pltpu.make_async_copy(dst3_hbm.at[pl.ds(chunk0, NCH)], didx3_v,
                          sem_g.at[1]).wait()
    for t in range(SEG // CH):
        pltpu.sync_copy(rows1, acc_sh.at[pl.ds(sid * SEG + t * CH, CH)])
    plsc.subcore_barrier()

    def group(gi, _):
        j0 = gi * NBUF
        consume(j0, 0, 1, False)
        consume(j0 + 1, 1, 0, False)
        return 0

    lax.fori_loop(0, NCH // NBUF, group, 0)
    consume(NCH - 1, 0, 1, True)
    pltpu.make_async_copy(rows[0], acc_sh.at[didx3_v.at[NCH - 1]],
                          sem_s.at[0]).wait()
    plsc.subcore_barrier()

    pltpu.sync_copy(acc_sh.at[pl.ds(sid * SEG, SEG)],
                    out_hbm.at[cid, pl.ds(sid * SEG, SEG)])


_msg_call = pl.kernel(
    _msg_body,
    out_type=jax.ShapeDtypeStruct((NC, NPAD, D), jnp.float32),
    mesh=plsc.VectorSubcoreMesh(**_MESH),
    compiler_params=_SC_PARAMS,
    scratch_types=(
        [pltpu.VMEM((EPTM,), jnp.int32),
         pltpu.VMEM((NCH, 1, CH), jnp.int32)]
        + [pltpu.VMEM((CH,), jnp.float32) for _ in range(NBUF)]
        + [pltpu.VMEM((CH, D), jnp.float32) for _ in range(NBUF)]
        + [pltpu.VMEM_SHARED((NPAD, D), jnp.float32),
           pltpu.SemaphoreType.DMA((NBUF,)),
           pltpu.SemaphoreType.DMA((NBUF,)),
           pltpu.SemaphoreType.DMA((NBUF,)),
           pltpu.SemaphoreType.DMA]
    ),
)


def _fin_body(p0_ref, p1_ref, r_ref, o_ref):
    o_ref[...] = (p0_ref[0] + p1_ref[0]) * r_ref[...]


_fin_call = pl.pallas_call(
    _fin_body,
    grid=(N // BR,),
    in_specs=[
        pl.BlockSpec((1, BR, D), lambda j: (0, j, 0)),
        pl.BlockSpec((1, BR, D), lambda j: (1, j, 0)),
        pl.BlockSpec((BR, 1), lambda j: (j, 0)),
    ],
    out_specs=pl.BlockSpec((BR, D), lambda j: (j, 0)),
    out_shape=jax.ShapeDtypeStruct((N, D), jnp.float32),
)


@jax.jit
def kernel(u_f, i_f, edge_index, e_f, Wu, bu, Wi, bi):
    src = edge_index[0]
    dst = edge_index[1]

    invs = _deg_call(src, dst)
    inv_out = invs[0, :N].reshape(N, 1)
    inv_in = invs[1, :N].reshape(N, 1)

    node = _lin_call(u_f, i_f, Wu.T, Wi.T, bu.reshape(1, D),
                     bi.reshape(1, D), inv_out)

    dst3 = dst.reshape(E // CH, 1, CH)
    parts = _msg_call(node, src, dst3, e_f)
    return _fin_call(parts, parts, inv_in)


# speedup vs baseline: 1.4363x; 1.4363x over previous
import functools

import jax
import jax.numpy as jnp
from jax import lax
from jax.experimental import pallas as pl
from jax.experimental.pallas import tpu as pltpu
from jax.experimental.pallas import tpu_sc as plsc

N_U = 5000
N_I = 5000
N = N_U + N_I
NPAD = 10240
E = 320000
D = 128
NC, NS = 2, 16
CH = 80
EPC = E // NC
EPTM = EPC // NS
NCH = EPTM // CH
EPT = E // NS
SEG = NPAD // NS
LANES = 16

_MESH = dict(core_axis_name="c", subcore_axis_name="s", num_cores=NC,
             num_subcores=NS)
_SC_PARAMS = pltpu.CompilerParams(needs_layout_passes=False)


def _rsqrt_newton(d):
    i = plsc.bitcast(d, jnp.int32)
    i = jnp.int32(0x5F3759DF) - lax.shift_right_logical(i, 1)
    y = plsc.bitcast(i, jnp.float32)
    for _ in range(3):
        y = y * (1.5 - 0.5 * d * y * y)
    return y


def _deg_body(src_hbm, dst_hbm, out_hbm, idx_v, hist_v, red_v, res_v,
              stage_sh, sem_p):
    cid = lax.axis_index("c")
    sid = lax.axis_index("s")
    zero16 = jnp.zeros((LANES,), jnp.float32)
    ones16 = jnp.ones((LANES,), jnp.float32)
    base = sid * EPT

    def _histogram(arr_hbm, h):
        pltpu.async_copy(arr_hbm.at[pl.ds(base, EPT)], idx_v, sem_p)

        def fill_zeros(i, _):
            hist_v[pl.ds(i * LANES, LANES)] = zero16
            return 0

        lax.fori_loop(0, NPAD // LANES, fill_zeros, 0)
        pltpu.make_async_copy(arr_hbm.at[pl.ds(base, EPT)], idx_v,
                              sem_p).wait()

        def hist(i, _):
            iv = idx_v[pl.ds(i * LANES, LANES)]
            plsc.addupdate_scatter(hist_v, [iv], ones16)
            return 0

        lax.fori_loop(0, EPT // LANES, hist, 0)

        pltpu.sync_copy(hist_v, stage_sh.at[sid])
        plsc.subcore_barrier()
        pltpu.sync_copy(stage_sh.at[:, pl.ds(sid * SEG, SEG)], red_v)

        def reduce(v, _):
            sl = pl.ds(v * LANES, LANES)
            acc = zero16
            for t in range(NS):
                acc = acc + red_v[t, sl]
            res_v[sl] = _rsqrt_newton(jnp.maximum(acc, 1.0))
            return 0

        lax.fori_loop(0, SEG // LANES, reduce, 0)
        pltpu.sync_copy(res_v, out_hbm.at[h, pl.ds(sid * SEG, SEG)])

    @pl.when(cid == 0)
    def _():
        _histogram(src_hbm, 0)

    @pl.when(cid == 1)
    def _():
        _histogram(dst_hbm, 1)


_deg_call = pl.kernel(
    _deg_body,
    out_type=jax.ShapeDtypeStruct((2, NPAD), jnp.float32),
    mesh=plsc.VectorSubcoreMesh(**_MESH),
    compiler_params=_SC_PARAMS,
    scratch_types=[
        pltpu.VMEM((EPT,), jnp.int32),
        pltpu.VMEM((NPAD,), jnp.float32),
        pltpu.VMEM((NS, SEG), jnp.float32),
        pltpu.VMEM((SEG,), jnp.float32),
        pltpu.VMEM_SHARED((NS, NPAD), jnp.float32),
        pltpu.SemaphoreType.DMA,
    ],
)


BR = 1000
NBU = N_U // BR


def _lin_body(u_ref, i_ref, wu_ref, wi_ref, bu_ref, bi_ref, r_ref, o_ref):
    j = pl.program_id(0)
    x = jnp.where(j < NBU, u_ref[...], i_ref[...])
    w = jnp.where(j < NBU, wu_ref[...], wi_ref[...])
    b = jnp.where(j < NBU, bu_ref[...], bi_ref[...])
    y = jnp.dot(x, w, preferred_element_type=jnp.float32) + b
    o_ref[...] = y * r_ref[...]


_lin_call = pl.pallas_call(
    _lin_body,
    grid=(N // BR,),
    in_specs=[
        pl.BlockSpec((BR, D), lambda j: (jnp.minimum(j, NBU - 1), 0)),
        pl.BlockSpec((BR, D), lambda j: (jnp.maximum(j - NBU, 0), 0)),
        pl.BlockSpec((D, D), lambda j: (0, 0)),
        pl.BlockSpec((D, D), lambda j: (0, 0)),
        pl.BlockSpec((1, D), lambda j: (0, 0)),
        pl.BlockSpec((1, D), lambda j: (0, 0)),
        pl.BlockSpec((BR, 1), lambda j: (j, 0)),
    ],
    out_specs=pl.BlockSpec((BR, D), lambda j: (j, 0)),
    out_shape=jax.ShapeDtypeStruct((N, D), jnp.float32),
)


NBUF = 3


def _msg_body(node_hbm, src_hbm, dst_hbm, ef_hbm, out_hbm,
              sidx_v,
              ef0, ef1, ef2,
              didx0, didx1, didx2,
              rows0, rows1, rows2,
              acc_sh, sem_g, sem_d, sem_e, sem_s, sem_p):
    efs = (ef0, ef1, ef2)
    didx = (didx0, didx1, didx2)
    rows = (rows0, rows1, rows2)
    cid = lax.axis_index("c")
    sid = lax.axis_index("s")
    base = cid * EPC + sid * EPTM

    pltpu.async_copy(src_hbm.at[pl.ds(base, EPTM)], sidx_v, sem_p)

    def issue(j, b):
        pltpu.async_copy(dst_hbm.at[pl.ds(base + j * CH, CH)],
                         didx[b], sem_d.at[b])
        pltpu.async_copy(ef_hbm.at[pl.ds(base + j * CH, CH)],
                         efs[b], sem_e.at[b])
        pltpu.async_copy(node_hbm.at[sidx_v.at[pl.ds(j * CH, CH)]],
                         rows[b], sem_g.at[b])

    def consume(j, b):
        pltpu.make_async_copy(node_hbm.at[sidx_v.at[pl.ds(j * CH, CH)]],
                              rows[b], sem_g.at[b]).wait()
        pltpu.make_async_copy(ef_hbm.at[pl.ds(base + j * CH, CH)],
                              efs[b], sem_e.at[b]).wait()

        @plsc.parallel_loop(0, CH, 1, unroll=4)
        def scale(i):
            e = plsc.load_gather(efs[b],
                                 [jnp.full((LANES,), i, jnp.int32)])
            for k in range(D // LANES):
                sl = pl.ds(k * LANES, LANES)
                rows[b][i, sl] = rows[b][i, sl] * e

        pltpu.make_async_copy(dst_hbm.at[pl.ds(base + j * CH, CH)],
                              didx[b], sem_d.at[b]).wait()
        pltpu.async_copy(rows[b], acc_sh.at[didx[b]], sem_s.at[b],
                         add=True)

    def zero_rows(i, _):
        for k in range(D // LANES):
            rows2[i, pl.ds(k * LANES, LANES)] = jnp.zeros((LANES,),
                                                          jnp.float32)
        return 0

    lax.fori_loop(0, CH, zero_rows, 0)
    pltpu.make_async_copy(src_hbm.at[pl.ds(base, EPTM)], sidx_v,
                          sem_p).wait()
    for b in range(NBUF - 1):
        issue(b, b)
    for t in range(SEG // CH):
        pltpu.sync_copy(rows2, acc_sh.at[pl.ds(sid * SEG + t * CH, CH)])
    plsc.subcore_barrier()

    def group(gi, _):
        j0 = gi * NBUF
        for b in range(NBUF):
            j = j0 + b
            consume(j, b)
            bn = (b + NBUF - 1) % NBUF
            jn = j + NBUF - 1

            @pl.when(jnp.logical_and(jn < NCH, jn >= NBUF))
            def _():
                pltpu.make_async_copy(rows[bn], acc_sh.at[didx[bn]],
                                      sem_s.at[bn]).wait()

            @pl.when(jn < NCH)
            def _():
                issue(jn, bn)

        return 0

    lax.fori_loop(0, NCH // NBUF, group, 0)
    for j in range((NCH // NBUF) * NBUF, NCH):
        consume(j, j % NBUF)
    for b in range(NBUF):
        pltpu.make_async_copy(rows[b], acc_sh.at[didx[b]],
                              sem_s.at[b]).wait()
    plsc.subcore_barrier()

    pltpu.sync_copy(acc_sh.at[pl.ds(sid * SEG, SEG)],
                    out_hbm.at[cid, pl.ds(sid * SEG, SEG)])


_msg_call = pl.kernel(
    _msg_body,
    out_type=jax.ShapeDtypeStruct((NC, NPAD, D), jnp.float32),
    mesh=plsc.VectorSubcoreMesh(**_MESH),
    compiler_params=_SC_PARAMS,
    scratch_types=(
        [pltpu.VMEM((EPTM,), jnp.int32)]
        + [pltpu.VMEM((CH,), jnp.float32) for _ in range(NBUF)]
        + [pltpu.VMEM((CH,), jnp.int32) for _ in range(NBUF)]
        + [pltpu.VMEM((CH, D), jnp.float32) for _ in range(NBUF)]
        + [pltpu.VMEM_SHARED((NPAD, D), jnp.float32),
           pltpu.SemaphoreType.DMA((NBUF,)),
           pltpu.SemaphoreType.DMA((NBUF,)),
           pltpu.SemaphoreType.DMA((NBUF,)),
           pltpu.SemaphoreType.DMA((NBUF,)),
           pltpu.SemaphoreType.DMA]
    ),
)


def _fin_body(p0_ref, p1_ref, r_ref, o_ref):
    o_ref[...] = (p0_ref[0] + p1_ref[0]) * r_ref[...]


_fin_call = pl.pallas_call(
    _fin_body,
    grid=(N // BR,),
    in_specs=[
        pl.BlockSpec((1, BR, D), lambda j: (0, j, 0)),
        pl.BlockSpec((1, BR, D), lambda j: (1, j, 0)),
        pl.BlockSpec((BR, 1), lambda j: (j, 0)),
    ],
    out_specs=pl.BlockSpec((BR, D), lambda j: (j, 0)),
    out_shape=jax.ShapeDtypeStruct((N, D), jnp.float32),
)


@jax.jit
def kernel(u_f, i_f, edge_index, e_f, Wu, bu, Wi, bi):
    src = edge_index[0]
    dst = edge_index[1]

    invs = _deg_call(src, dst)
    inv_out = invs[0, :N].reshape(N, 1)
    inv_in = invs[1, :N].reshape(N, 1)

    node = _lin_call(u_f, i_f, Wu.T, Wi.T, bu.reshape(1, D),
                     bi.reshape(1, D), inv_out)

    parts = _msg_call(node, src, dst, e_f)
    return _fin_call(parts, parts, inv_in)
